# baseline (speedup 1.0000x reference)
import functools

import jax
import jax.numpy as jnp
from jax import lax
from jax.experimental import pallas as pl
from jax.experimental.pallas import tpu as pltpu
from jax.experimental.pallas import tpu_sc as plsc

N = 10000
E = 160000
G = 64
C = 36

NC = 2
NS = 16
K = 80
CH = 64
E_PAD = NC * NS * CH * K
NPAD = 10240
ZCH = NPAD // NS // K
STRIPE = 624
TAIL = N - NS * STRIPE
OCH = [K] * 7 + [STRIPE - 7 * K]

BR = 2000


def _sc_segment_sum(F, stage_y=False, col_split=False):
  nch = 2 * CH if col_split else CH
  stage_y = stage_y or col_split
  mesh = plsc.VectorSubcoreMesh(core_axis_name="c", subcore_axis_name="s",
                                num_cores=NC, num_subcores=NS)
  scratch = [
      pltpu.VMEM((nch * K,), jnp.int32),
      pltpu.VMEM((nch, K), jnp.int32),
      pltpu.VMEM((K, F), jnp.float32),
      pltpu.VMEM((K, F), jnp.float32),
      pltpu.VMEM_SHARED((NPAD, F), jnp.float32),
      pltpu.SemaphoreType.DMA,
      pltpu.SemaphoreType.DMA,
  ]
  if stage_y:
    scratch.append(pltpu.VMEM_SHARED((N, F), jnp.float32))

  @functools.partial(
      pl.kernel,
      out_type=jax.ShapeDtypeStruct((2 * N, F), jnp.float32),
      mesh=mesh,
      scratch_types=scratch,
      compiler_params=pltpu.CompilerParams(use_tc_tiling_on_sc=False),
  )
  def sc(y_hbm, srcf_hbm, dstm_hbm, zer_hbm, out_hbm,
         src_v, dst_v, rows0_v, rows1_v, acc_sh, sem0, sem1, *maybe_ysh):
    cid = lax.axis_index("c")
    sid = lax.axis_index("s")
    wid = sid if col_split else cid * NS + sid
    pltpu.sync_copy(srcf_hbm.at[pl.ds(wid * nch * K, nch * K)], src_v)
    pltpu.sync_copy(dstm_hbm.at[wid], dst_v)
    if stage_y:
      ysh = maybe_ysh[0]
      ybase = cid * N if col_split else 0
      off = 0
      for w in OCH:
        r0 = sid * STRIPE + off
        pltpu.sync_copy(y_hbm.at[pl.ds(ybase + r0, w)], rows1_v.at[pl.ds(0, w)])
        pltpu.sync_copy(rows1_v.at[pl.ds(0, w)], ysh.at[pl.ds(r0, w)])
        off += w

      @pl.when(sid == NS - 1)
      def _():
        t0 = NS * STRIPE
        pltpu.sync_copy(y_hbm.at[pl.ds(ybase + t0, TAIL)],
                        rows1_v.at[pl.ds(0, TAIL)])
        pltpu.sync_copy(rows1_v.at[pl.ds(0, TAIL)], ysh.at[pl.ds(t0, TAIL)])

      ysrc = ysh
    else:
      ysrc = y_hbm
    pltpu.sync_copy(zer_hbm, rows0_v)
    for z in range(ZCH):
      pltpu.sync_copy(rows0_v, acc_sh.at[pl.ds(sid * (ZCH * K) + z * K, K)])
    plsc.subcore_barrier()

    def sidx(c):
      return src_v.at[pl.ds(c * K, K)]

    pltpu.async_copy(ysrc.at[sidx(0)], rows0_v, sem0)

    def body(c2, carry):
      c = 2 * c2
      pltpu.make_async_copy(ysrc.at[sidx(c)], rows0_v, sem0).wait()
      pltpu.async_copy(ysrc.at[sidx(c + 1)], rows1_v, sem1)
      pltpu.sync_copy(rows0_v, acc_sh.at[dst_v.at[c]], add=True)
      pltpu.make_async_copy(ysrc.at[sidx(c + 1)], rows1_v, sem1).wait()

      @pl.when(c + 2 < nch)
      def _():
        pltpu.async_copy(ysrc.at[sidx(c + 2)], rows0_v, sem0)

      pltpu.sync_copy(rows1_v, acc_sh.at[dst_v.at[c + 1]], add=True)
      return carry

    lax.fori_loop(0, nch // 2, body, 0)
    plsc.subcore_barrier()
    off = 0
    for w in OCH:
      r0 = sid * STRIPE + off
      pltpu.sync_copy(acc_sh.at[pl.ds(r0, w)], rows0_v.at[pl.ds(0, w)])
      pltpu.sync_copy(rows0_v.at[pl.ds(0, w)],
                      out_hbm.at[pl.ds(cid * N + r0, w)])
      off += w

    @pl.when(sid == NS - 1)
    def _():
      t0 = NS * STRIPE
      pltpu.sync_copy(acc_sh.at[pl.ds(t0, TAIL)], rows0_v.at[pl.ds(0, TAIL)])
      pltpu.sync_copy(rows0_v.at[pl.ds(0, TAIL)],
                      out_hbm.at[pl.ds(cid * N + t0, TAIL)])

  return sc


def _mm_body(x_ref, wr_ref, wt_ref, y_ref, r_ref):
  xb = x_ref[...]
  dn = (((1,), (1,)), ((), ()))
  y_ref[...] = lax.dot_general(xb, wr_ref[...], dn,
                               preferred_element_type=jnp.float32)
  r_ref[...] = lax.dot_general(xb, wt_ref[...], dn,
                               preferred_element_type=jnp.float32)


def _mm(x, w_rel, w_root):
  fin = x.shape[1]
  fout = w_rel.shape[0]
  grid = N // BR
  return pl.pallas_call(
      _mm_body,
      grid=(grid,),
      in_specs=[
          pl.BlockSpec((BR, fin), lambda i: (i, 0)),
          pl.BlockSpec((fout, fin), lambda i: (0, 0)),
          pl.BlockSpec((fout, fin), lambda i: (0, 0)),
      ],
      out_specs=[
          pl.BlockSpec((BR, fout), lambda i: (i, 0)),
          pl.BlockSpec((BR, fout), lambda i: (i, 0)),
      ],
      out_shape=[
          jax.ShapeDtypeStruct((N, fout), jnp.float32),
          jax.ShapeDtypeStruct((N, fout), jnp.float32),
      ],
  )(x, w_rel, w_root)


def _mm_split_body(x_ref, wr_ref, wt_ref, y_ref, r_ref):
  xb = x_ref[...]
  dn = (((1,), (1,)), ((), ()))
  y_ref[...] = lax.dot_general(xb, wr_ref[0], dn,
                               preferred_element_type=jnp.float32)
  r_ref[...] = lax.dot_general(xb, wt_ref[0], dn,
                               preferred_element_type=jnp.float32)


def _mm_split(x, w_rel, w_root):
  fin = x.shape[1]
  fh = w_rel.shape[0] // 2
  grid = (2, N // BR)
  return pl.pallas_call(
      _mm_split_body,
      grid=grid,
      in_specs=[
          pl.BlockSpec((BR, fin), lambda h, i: (i, 0)),
          pl.BlockSpec((1, fh, fin), lambda h, i: (h, 0, 0)),
          pl.BlockSpec((1, fh, fin), lambda h, i: (h, 0, 0)),
      ],
      out_specs=[
          pl.BlockSpec((BR, fh), lambda h, i: (h * (N // BR) + i, 0)),
          pl.BlockSpec((BR, fh), lambda h, i: (h * (N // BR) + i, 0)),
      ],
      out_shape=[
          jax.ShapeDtypeStruct((2 * N, fh), jnp.float32),
          jax.ShapeDtypeStruct((2 * N, fh), jnp.float32),
      ],
  )(x, w_rel.reshape(2, fh, fin), w_root.reshape(2, fh, fin))


def _stats_cat_body(p0_ref, p1_ref, r0_ref, r1_ref, z_ref, s_ref):
  i = pl.program_id(0)
  z = jnp.concatenate(
      [p0_ref[...] + r0_ref[...], p1_ref[...] + r1_ref[...]], axis=1)
  z_ref[...] = z
  cs = jnp.sum(z, axis=0, keepdims=True)
  cq = jnp.sum(z * z, axis=0, keepdims=True)
  blk = jnp.concatenate(
      [cs, cq, jnp.zeros((6, z.shape[1]), jnp.float32)], axis=0)

  @pl.when(i == 0)
  def _():
    s_ref[...] = blk

  @pl.when(i != 0)
  def _():
    s_ref[...] = s_ref[...] + blk


def _stats_cat(p, r):
  fh = r.shape[1]
  grid = N // BR
  half = pl.BlockSpec((BR, fh), lambda i: (i, 0))
  half2 = pl.BlockSpec((BR, fh), lambda i: (i + N // BR, 0))
  return pl.pallas_call(
      _stats_cat_body,
      grid=(grid,),
      in_specs=[half, half2, half, half2],
      out_specs=[
          pl.BlockSpec((BR, 2 * fh), lambda i: (i, 0)),
          pl.BlockSpec((8, 2 * fh), lambda i: (0, 0)),
      ],
      out_shape=[
          jax.ShapeDtypeStruct((N, 2 * fh), jnp.float32),
          jax.ShapeDtypeStruct((8, 2 * fh), jnp.float32),
      ],
  )(p, p, r, r)


def _stats_body(p0_ref, p1_ref, r_ref, z_ref, s_ref):
  i = pl.program_id(0)
  z = p0_ref[...] + p1_ref[...] + r_ref[...]
  z_ref[...] = z
  cs = jnp.sum(z, axis=0, keepdims=True)
  cq = jnp.sum(z * z, axis=0, keepdims=True)
  blk = jnp.concatenate(
      [cs, cq, jnp.zeros((6, z.shape[1]), jnp.float32)], axis=0)

  @pl.when(i == 0)
  def _():
    s_ref[...] = blk

  @pl.when(i != 0)
  def _():
    s_ref[...] = s_ref[...] + blk


def _stats(p, r):
  f = r.shape[1]
  grid = N // BR
  return pl.pallas_call(
      _stats_body,
      grid=(grid,),
      in_specs=[
          pl.BlockSpec((BR, f), lambda i: (i, 0)),
          pl.BlockSpec((BR, f), lambda i: (i + N // BR, 0)),
          pl.BlockSpec((BR, f), lambda i: (i, 0)),
      ],
      out_specs=[
          pl.BlockSpec((BR, f), lambda i: (i, 0)),
          pl.BlockSpec((8, f), lambda i: (0, 0)),
      ],
      out_shape=[
          jax.ShapeDtypeStruct((N, f), jnp.float32),
          jax.ShapeDtypeStruct((8, f), jnp.float32),
      ],
  )(p, p, r)


def _bnmm_body(z_ref, s_ref, g_ref, be_ref, wr_ref, wt_ref, y_ref, r_ref):
  s = s_ref[...]
  mu = s[0:1, :] * (1.0 / N)
  var = s[1:2, :] * (1.0 / N) - mu * mu
  scale = g_ref[...] * lax.rsqrt(var + 1e-5)
  shift = be_ref[...] - mu * scale
  h = jnp.maximum(z_ref[...] * scale + shift, 0.0)
  dn = (((1,), (1,)), ((), ()))
  y_ref[...] = lax.dot_general(h, wr_ref[...], dn,
                               preferred_element_type=jnp.float32)
  r_ref[...] = lax.dot_general(h, wt_ref[...], dn,
                               preferred_element_type=jnp.float32)


def _bnmm(z, s, g, be, w_rel, w_root):
  fin = z.shape[1]
  fout = w_rel.shape[0]
  grid = N // BR
  return pl.pallas_call(
      _bnmm_body,
      grid=(grid,),
      in_specs=[
          pl.BlockSpec((BR, fin), lambda i: (i, 0)),
          pl.BlockSpec((8, fin), lambda i: (0, 0)),
          pl.BlockSpec((1, fin), lambda i: (0, 0)),
          pl.BlockSpec((1, fin), lambda i: (0, 0)),
          pl.BlockSpec((fout, fin), lambda i: (0, 0)),
          pl.BlockSpec((fout, fin), lambda i: (0, 0)),
      ],
      out_specs=[
          pl.BlockSpec((BR, fout), lambda i: (i, 0)),
          pl.BlockSpec((BR, fout), lambda i: (i, 0)),
      ],
      out_shape=[
          jax.ShapeDtypeStruct((N, fout), jnp.float32),
          jax.ShapeDtypeStruct((N, fout), jnp.float32),
      ],
  )(z, s, g.reshape(1, fin), be.reshape(1, fin), w_rel, w_root)


def _final_body(p0_ref, p1_ref, r_ref, b3_ref, bt_ref, wl_ref, bl_ref,
                out_ref, acc_ref):
  i = pl.program_id(0)

  @pl.when(i == 0)
  def _():
    acc_ref[...] = jnp.zeros_like(acc_ref)

  h = jnp.maximum(p0_ref[...] + p1_ref[...] + r_ref[...] + b3_ref[...], 0.0)
  bt = bt_ref[0, 0, :]
  oh = (lax.broadcasted_iota(jnp.int32, (G, BR), 0) == bt[None, :]).astype(
      jnp.float32)
  hcat = jnp.concatenate(
      [h, jnp.ones((BR, 1), jnp.float32), jnp.zeros((BR, 31), jnp.float32)],
      axis=1)
  acc_ref[...] += lax.dot_general(oh, hcat, (((1,), (0,)), ((), ())),
                                  preferred_element_type=jnp.float32)

  @pl.when(i == pl.num_programs(0) - 1)
  def _():
    acc = acc_ref[...]
    pooled = acc[:, :32] / jnp.maximum(acc[:, 32:33], 1.0)
    logits = lax.dot_general(pooled, wl_ref[...], (((1,), (1,)), ((), ())),
                             preferred_element_type=jnp.float32)
    logits = logits + bl_ref[...]
    m = jnp.max(logits, axis=1, keepdims=True)
    e = jnp.exp(logits - m)
    out_ref[...] = logits - m - jnp.log(jnp.sum(e, axis=1, keepdims=True))


def _final(p, r3, b3, batch3, wl, bl):
  grid = N // BR
  return pl.pallas_call(
      _final_body,
      grid=(grid,),
      in_specs=[
          pl.BlockSpec((BR, 32), lambda i: (i, 0)),
          pl.BlockSpec((BR, 32), lambda i: (i + N // BR, 0)),
          pl.BlockSpec((BR, 32), lambda i: (i, 0)),
          pl.BlockSpec((1, 32), lambda i: (0, 0)),
          pl.BlockSpec((1, 1, BR), lambda i: (i, 0, 0)),
          pl.BlockSpec((C, 32), lambda i: (0, 0)),
          pl.BlockSpec((1, C), lambda i: (0, 0)),
      ],
      out_specs=pl.BlockSpec((G, C), lambda i: (0, 0)),
      out_shape=jax.ShapeDtypeStruct((G, C), jnp.float32),
      scratch_shapes=[pltpu.VMEM((G, 64), jnp.float32)],
  )(p, p, r3, b3.reshape(1, 32), batch3, wl, bl.reshape(1, C))


def kernel(x, edge_index, batch, W1_rel, b1, W1_root, g1, be1, W2_rel, b2,
           W2_root, g2, be2, W3_rel, b3, W3_root, Wl, bl):
  src = edge_index[0]
  dst = edge_index[1]
  pad = E_PAD - E
  srcf = jnp.concatenate([src, jnp.zeros((pad,), jnp.int32)])
  dstm = jnp.concatenate([dst, jnp.full((pad,), N, jnp.int32)]).reshape(
      NC * NS, CH, K)

  dstm16 = dstm.reshape(NS, 2 * CH, K)

  sc64cs = _sc_segment_sum(64, col_split=True)
  sc64 = _sc_segment_sum(64, stage_y=True)
  sc32 = _sc_segment_sum(32, stage_y=True)
  z64 = jnp.zeros((K, 64), jnp.float32)
  z32 = jnp.zeros((K, 32), jnp.float32)

  ycat, rcat = _mm_split(x, W1_rel, W1_root)
  p1 = sc64cs(ycat, srcf, dstm16, z64)
  zz1, s1 = _stats_cat(p1, rcat)
  y2, r2 = _bnmm(zz1, s1, g1, be1, W2_rel, W2_root)
  p2 = sc64(y2, srcf, dstm, z64)
  zz2, s2 = _stats(p2, r2)
  y3, r3 = _bnmm(zz2, s2, g2, be2, W3_rel, W3_root)
  p3 = sc32(y3, srcf, dstm, z32)
  batch3 = batch.reshape(N // BR, 1, BR)
  return _final(p3, r3, b3, batch3, Wl, bl)


# speedup vs baseline: 1.0766x; 1.0766x over previous
import functools

import jax
import jax.numpy as jnp
from jax import lax
from jax.experimental import pallas as pl
from jax.experimental.pallas import tpu as pltpu
from jax.experimental.pallas import tpu_sc as plsc

N = 10000
E = 160000
G = 64
C = 36

NC = 2
NS = 16
K = 128
CH = 40
E_PAD = NC * NS * CH * K
NPAD = 10240
ZCH = NPAD // NS // K
STRIPE = 624
TAIL = N - NS * STRIPE
OCH = [K] * 4 + [STRIPE - 4 * K]

BR = 2000


def _sc_segment_sum(F, stage_y=False, col_split=False):
  nch = 2 * CH if col_split else CH
  stage_y = stage_y or col_split
  mesh = plsc.VectorSubcoreMesh(core_axis_name="c", subcore_axis_name="s",
                                num_cores=NC, num_subcores=NS)
  scratch = [
      pltpu.VMEM((nch * K,), jnp.int32),
      pltpu.VMEM((nch, K), jnp.int32),
      pltpu.VMEM((K, F), jnp.float32),
      pltpu.VMEM((K, F), jnp.float32),
      pltpu.VMEM_SHARED((NPAD, F), jnp.float32),
      pltpu.SemaphoreType.DMA,
      pltpu.SemaphoreType.DMA,
  ]
  if stage_y:
    scratch.append(pltpu.VMEM_SHARED((N, F), jnp.float32))

  def body_fn(ys, srcf_hbm, dstm_hbm, zer_hbm, out_hbm,
              src_v, dst_v, rows0_v, rows1_v, acc_sh, sem0, sem1, maybe_ysh):
    cid = lax.axis_index("c")
    sid = lax.axis_index("s")
    wid = sid if col_split else cid * NS + sid
    pltpu.sync_copy(srcf_hbm.at[pl.ds(wid * nch * K, nch * K)], src_v)
    pltpu.sync_copy(dstm_hbm.at[wid], dst_v)
    if stage_y:
      ysh = maybe_ysh[0]

      def stage_from(src_hbm):
        off = 0
        for w in OCH:
          r0 = sid * STRIPE + off
          pltpu.sync_copy(src_hbm.at[pl.ds(r0, w)], rows1_v.at[pl.ds(0, w)])
          pltpu.sync_copy(rows1_v.at[pl.ds(0, w)], ysh.at[pl.ds(r0, w)])
          off += w

        @pl.when(sid == NS - 1)
        def _():
          t0 = NS * STRIPE
          pltpu.sync_copy(src_hbm.at[pl.ds(t0, TAIL)],
                          rows1_v.at[pl.ds(0, TAIL)])
          pltpu.sync_copy(rows1_v.at[pl.ds(0, TAIL)], ysh.at[pl.ds(t0, TAIL)])

      if col_split:
        @pl.when(cid == 0)
        def _():
          stage_from(ys[0])

        @pl.when(cid == 1)
        def _():
          stage_from(ys[1])
      else:
        stage_from(ys[0])

      ysrc = ysh
    else:
      ysrc = ys[0]
    pltpu.sync_copy(zer_hbm, rows0_v)
    for z in range(ZCH):
      pltpu.sync_copy(rows0_v, acc_sh.at[pl.ds(sid * (ZCH * K) + z * K, K)])
    plsc.subcore_barrier()

    def sidx(c):
      return src_v.at[pl.ds(c * K, K)]

    pltpu.async_copy(ysrc.at[sidx(0)], rows0_v, sem0)

    def body(c2, carry):
      c = 2 * c2
      pltpu.make_async_copy(ysrc.at[sidx(c)], rows0_v, sem0).wait()
      pltpu.async_copy(ysrc.at[sidx(c + 1)], rows1_v, sem1)
      pltpu.sync_copy(rows0_v, acc_sh.at[dst_v.at[c]], add=True)
      pltpu.make_async_copy(ysrc.at[sidx(c + 1)], rows1_v, sem1).wait()

      @pl.when(c + 2 < nch)
      def _():
        pltpu.async_copy(ysrc.at[sidx(c + 2)], rows0_v, sem0)

      pltpu.sync_copy(rows1_v, acc_sh.at[dst_v.at[c + 1]], add=True)
      return carry

    lax.fori_loop(0, nch // 2, body, 0)
    plsc.subcore_barrier()
    off = 0
    for w in OCH:
      r0 = sid * STRIPE + off
      pltpu.sync_copy(acc_sh.at[pl.ds(r0, w)], rows0_v.at[pl.ds(0, w)])
      pltpu.sync_copy(rows0_v.at[pl.ds(0, w)],
                      out_hbm.at[pl.ds(cid * N + r0, w)])
      off += w

    @pl.when(sid == NS - 1)
    def _():
      t0 = NS * STRIPE
      pltpu.sync_copy(acc_sh.at[pl.ds(t0, TAIL)], rows0_v.at[pl.ds(0, TAIL)])
      pltpu.sync_copy(rows0_v.at[pl.ds(0, TAIL)],
                      out_hbm.at[pl.ds(cid * N + t0, TAIL)])

  kern = functools.partial(
      pl.kernel,
      out_type=jax.ShapeDtypeStruct((2 * N, F), jnp.float32),
      mesh=mesh,
      scratch_types=scratch,
      compiler_params=pltpu.CompilerParams(use_tc_tiling_on_sc=False),
  )

  if col_split:
    @kern
    def sc(ya, yb, *rest):
      body_fn((ya, yb), *rest[:11], rest[11:])
  else:
    @kern
    def sc(y, *rest):
      body_fn((y,), *rest[:11], rest[11:])

  return sc


def _mm4_body(x_ref, wr_ref, wt_ref, ya_ref, yb_ref, ra_ref, rb_ref):
  xb = x_ref[...]
  dn = (((1,), (1,)), ((), ()))
  ya_ref[...] = lax.dot_general(xb, wr_ref[0], dn,
                                preferred_element_type=jnp.float32)
  yb_ref[...] = lax.dot_general(xb, wr_ref[1], dn,
                                preferred_element_type=jnp.float32)
  ra_ref[...] = lax.dot_general(xb, wt_ref[0], dn,
                                preferred_element_type=jnp.float32)
  rb_ref[...] = lax.dot_general(xb, wt_ref[1], dn,
                                preferred_element_type=jnp.float32)


def _mm4(x, w_rel, w_root):
  fin = x.shape[1]
  fh = w_rel.shape[0] // 2
  grid = N // BR
  row = pl.BlockSpec((BR, fh), lambda i: (i, 0))
  wsp = pl.BlockSpec((2, fh, fin), lambda i: (0, 0, 0))
  osh = jax.ShapeDtypeStruct((N, fh), jnp.float32)
  return pl.pallas_call(
      _mm4_body,
      grid=(grid,),
      in_specs=[pl.BlockSpec((BR, fin), lambda i: (i, 0)), wsp, wsp],
      out_specs=[row, row, row, row],
      out_shape=[osh, osh, osh, osh],
  )(x, w_rel.reshape(2, fh, fin), w_root.reshape(2, fh, fin))


def _bn_tail(z, i, zbuf, s_acc):
  zbuf[pl.ds(i * BR, BR), :] = z
  cs = jnp.sum(z, axis=0, keepdims=True)
  cq = jnp.sum(z * z, axis=0, keepdims=True)
  blk = jnp.concatenate([cs, cq, jnp.zeros((6, z.shape[1]), jnp.float32)],
                        axis=0)

  @pl.when(i == 0)
  def _():
    s_acc[...] = blk

  @pl.when(i != 0)
  def _():
    s_acc[...] = s_acc[...] + blk


def _bn_mm_phase1(i, g_ref, be_ref, wr_ref, wt_ref, y_ref, r_ref,
                  zbuf, s_acc):
  s = s_acc[...]
  mu = s[0:1, :] * (1.0 / N)
  var = s[1:2, :] * (1.0 / N) - mu * mu
  scale = g_ref[...] * lax.rsqrt(var + 1e-5)
  shift = be_ref[...] - mu * scale
  h = jnp.maximum(zbuf[pl.ds(i * BR, BR), :] * scale + shift, 0.0)
  dn = (((1,), (1,)), ((), ()))
  y_ref[...] = lax.dot_general(h, wr_ref[...], dn,
                               preferred_element_type=jnp.float32)
  r_ref[...] = lax.dot_general(h, wt_ref[...], dn,
                               preferred_element_type=jnp.float32)


def _fused_cat_body(p0_ref, p1_ref, ra_ref, rb_ref, g_ref, be_ref,
                    wr_ref, wt_ref, y_ref, r_ref, zbuf, s_acc):
  t = pl.program_id(0)
  i = pl.program_id(1)

  @pl.when(t == 0)
  def _():
    z = jnp.concatenate(
        [p0_ref[...] + ra_ref[...], p1_ref[...] + rb_ref[...]], axis=1)
    _bn_tail(z, i, zbuf, s_acc)

  @pl.when(t == 1)
  def _():
    _bn_mm_phase1(i, g_ref, be_ref, wr_ref, wt_ref, y_ref, r_ref,
                  zbuf, s_acc)


def _fused_cat(p, ra, rb, g, be, w_rel, w_root):
  fin = 2 * ra.shape[1]
  fh = ra.shape[1]
  fout = w_rel.shape[0]
  nb = N // BR
  lo = pl.BlockSpec((BR, fh), lambda t, i: (i * (1 - t), 0))
  hi = pl.BlockSpec((BR, fh), lambda t, i: (i * (1 - t) + nb * (1 - t), 0))
  row = pl.BlockSpec((BR, fh), lambda t, i: (i * (1 - t), 0))
  vec = lambda f: pl.BlockSpec((1, f), lambda t, i: (0, 0))
  wsp = pl.BlockSpec((fout, fin), lambda t, i: (0, 0))
  orow = pl.BlockSpec((BR, fout), lambda t, i: (i * t, 0))
  return pl.pallas_call(
      _fused_cat_body,
      grid=(2, nb),
      in_specs=[lo, hi, row, row, vec(fin), vec(fin), wsp, wsp],
      out_specs=[orow, orow],
      out_shape=[
          jax.ShapeDtypeStruct((N, fout), jnp.float32),
          jax.ShapeDtypeStruct((N, fout), jnp.float32),
      ],
      scratch_shapes=[
          pltpu.VMEM((N, fin), jnp.float32),
          pltpu.VMEM((8, fin), jnp.float32),
      ],
  )(p, p, ra, rb, g.reshape(1, fin), be.reshape(1, fin), w_rel, w_root)


def _fused_sum_body(p0_ref, p1_ref, r_ref, g_ref, be_ref,
                    wr_ref, wt_ref, y_ref, r_out_ref, zbuf, s_acc):
  t = pl.program_id(0)
  i = pl.program_id(1)

  @pl.when(t == 0)
  def _():
    z = p0_ref[...] + p1_ref[...] + r_ref[...]
    _bn_tail(z, i, zbuf, s_acc)

  @pl.when(t == 1)
  def _():
    _bn_mm_phase1(i, g_ref, be_ref, wr_ref, wt_ref, y_ref, r_out_ref,
                  zbuf, s_acc)


def _fused_sum(p, r, g, be, w_rel, w_root):
  fin = r.shape[1]
  fout = w_rel.shape[0]
  nb = N // BR
  lo = pl.BlockSpec((BR, fin), lambda t, i: (i * (1 - t), 0))
  hi = pl.BlockSpec((BR, fin), lambda t, i: (i * (1 - t) + nb * (1 - t), 0))
  row = pl.BlockSpec((BR, fin), lambda t, i: (i * (1 - t), 0))
  vec = pl.BlockSpec((1, fin), lambda t, i: (0, 0))
  wsp = pl.BlockSpec((fout, fin), lambda t, i: (0, 0))
  orow = pl.BlockSpec((BR, fout), lambda t, i: (i * t, 0))
  return pl.pallas_call(
      _fused_sum_body,
      grid=(2, nb),
      in_specs=[lo, hi, row, vec, vec, wsp, wsp],
      out_specs=[orow, orow],
      out_shape=[
          jax.ShapeDtypeStruct((N, fout), jnp.float32),
          jax.ShapeDtypeStruct((N, fout), jnp.float32),
      ],
      scratch_shapes=[
          pltpu.VMEM((N, fin), jnp.float32),
          pltpu.VMEM((8, fin), jnp.float32),
      ],
  )(p, p, r, g.reshape(1, fin), be.reshape(1, fin), w_rel, w_root)


def _final_body(p0_ref, p1_ref, r_ref, b3_ref, bt_ref, wl_ref, bl_ref,
                out_ref, acc_ref):
  i = pl.program_id(0)

  @pl.when(i == 0)
  def _():
    acc_ref[...] = jnp.zeros_like(acc_ref)

  h = jnp.maximum(p0_ref[...] + p1_ref[...] + r_ref[...] + b3_ref[...], 0.0)
  bt = bt_ref[0, 0, :]
  oh = (lax.broadcasted_iota(jnp.int32, (G, BR), 0) == bt[None, :]).astype(
      jnp.float32)
  hcat = jnp.concatenate(
      [h, jnp.ones((BR, 1), jnp.float32), jnp.zeros((BR, 31), jnp.float32)],
      axis=1)
  acc_ref[...] += lax.dot_general(oh, hcat, (((1,), (0,)), ((), ())),
                                  preferred_element_type=jnp.float32)

  @pl.when(i == pl.num_programs(0) - 1)
  def _():
    acc = acc_ref[...]
    pooled = acc[:, :32] / jnp.maximum(acc[:, 32:33], 1.0)
    logits = lax.dot_general(pooled, wl_ref[...], (((1,), (1,)), ((), ())),
                             preferred_element_type=jnp.float32)
    logits = logits + bl_ref[...]
    m = jnp.max(logits, axis=1, keepdims=True)
    e = jnp.exp(logits - m)
    out_ref[...] = logits - m - jnp.log(jnp.sum(e, axis=1, keepdims=True))


def _final(p, r3, b3, batch3, wl, bl):
  grid = N // BR
  return pl.pallas_call(
      _final_body,
      grid=(grid,),
      in_specs=[
          pl.BlockSpec((BR, 32), lambda i: (i, 0)),
          pl.BlockSpec((BR, 32), lambda i: (i + N // BR, 0)),
          pl.BlockSpec((BR, 32), lambda i: (i, 0)),
          pl.BlockSpec((1, 32), lambda i: (0, 0)),
          pl.BlockSpec((1, 1, BR), lambda i: (i, 0, 0)),
          pl.BlockSpec((C, 32), lambda i: (0, 0)),
          pl.BlockSpec((1, C), lambda i: (0, 0)),
      ],
      out_specs=pl.BlockSpec((G, C), lambda i: (0, 0)),
      out_shape=jax.ShapeDtypeStruct((G, C), jnp.float32),
      scratch_shapes=[pltpu.VMEM((G, 64), jnp.float32)],
  )(p, p, r3, b3.reshape(1, 32), batch3, wl, bl.reshape(1, C))


def kernel(x, edge_index, batch, W1_rel, b1, W1_root, g1, be1, W2_rel, b2,
           W2_root, g2, be2, W3_rel, b3, W3_root, Wl, bl):
  src = edge_index[0]
  dst = edge_index[1]
  pad = E_PAD - E
  srcf = jnp.concatenate([src, jnp.zeros((pad,), jnp.int32)])
  dstm = jnp.concatenate([dst, jnp.full((pad,), N, jnp.int32)]).reshape(
      NC * NS, CH, K)

  dstm16 = dstm.reshape(NS, 2 * CH, K)

  sc64cs = _sc_segment_sum(64, col_split=True)
  sc64 = _sc_segment_sum(64, stage_y=True)
  sc32 = _sc_segment_sum(32, stage_y=True)
  z64 = jnp.zeros((K, 64), jnp.float32)
  z32 = jnp.zeros((K, 32), jnp.float32)

  ya, yb, ra, rb = _mm4(x, W1_rel, W1_root)
  p1 = sc64cs(ya, yb, srcf, dstm16, z64)
  y2, r2 = _fused_cat(p1, ra, rb, g1, be1, W2_rel, W2_root)
  p2 = sc64(y2, srcf, dstm, z64)
  y3, r3 = _fused_sum(p2, r2, g2, be2, W3_rel, W3_root)
  p3 = sc32(y3, srcf, dstm, z32)
  batch3 = batch.reshape(N // BR, 1, BR)
  return _final(p3, r3, b3, batch3, Wl, bl)


# speedup vs baseline: 1.2003x; 1.1150x over previous
import functools

import jax
import jax.numpy as jnp
from jax import lax
from jax.experimental import pallas as pl
from jax.experimental.pallas import tpu as pltpu
from jax.experimental.pallas import tpu_sc as plsc

N = 10000
E = 160000
G = 64
C = 36

NC = 2
NS = 16
K = 128
CH = 40
E_PAD = NC * NS * CH * K
NPAD = 10240
ZCH = NPAD // NS // K
STRIPE = 624
TAIL = N - NS * STRIPE
OCH = [K] * 4 + [STRIPE - 4 * K]

BR = 2000


def _sc_segment_sum(F, stage_y=False, col_split=False):
  nch = 2 * CH if col_split else CH
  stage_y = stage_y or col_split
  mesh = plsc.VectorSubcoreMesh(core_axis_name="c", subcore_axis_name="s",
                                num_cores=NC, num_subcores=NS)
  scratch = [
      pltpu.VMEM((nch * K,), jnp.int32),
      pltpu.VMEM((nch, K), jnp.int32),
      pltpu.VMEM((K, F), jnp.float32),
      pltpu.VMEM((K, F), jnp.float32),
      pltpu.VMEM_SHARED((NPAD, F), jnp.float32),
      pltpu.SemaphoreType.DMA,
      pltpu.SemaphoreType.DMA,
  ]
  if stage_y:
    scratch.append(pltpu.VMEM_SHARED((N, F), jnp.float32))

  def body_fn(ys, srcf_hbm, dstm_hbm, zer_hbm, out_hbm,
              src_v, dst_v, rows0_v, rows1_v, acc_sh, sem0, sem1, maybe_ysh):
    cid = lax.axis_index("c")
    sid = lax.axis_index("s")
    wid = sid if col_split else cid * NS + sid
    pltpu.sync_copy(srcf_hbm.at[pl.ds(wid * nch * K, nch * K)], src_v)
    pltpu.sync_copy(dstm_hbm.at[wid], dst_v)
    if stage_y:
      ysh = maybe_ysh[0]

      def stage_from(src_hbm):
        off = 0
        for w in OCH:
          r0 = sid * STRIPE + off
          pltpu.sync_copy(src_hbm.at[pl.ds(r0, w)], rows1_v.at[pl.ds(0, w)])
          pltpu.sync_copy(rows1_v.at[pl.ds(0, w)], ysh.at[pl.ds(r0, w)])
          off += w

        @pl.when(sid == NS - 1)
        def _():
          t0 = NS * STRIPE
          pltpu.sync_copy(src_hbm.at[pl.ds(t0, TAIL)],
                          rows1_v.at[pl.ds(0, TAIL)])
          pltpu.sync_copy(rows1_v.at[pl.ds(0, TAIL)], ysh.at[pl.ds(t0, TAIL)])

      if col_split:
        @pl.when(cid == 0)
        def _():
          stage_from(ys[0])

        @pl.when(cid == 1)
        def _():
          stage_from(ys[1])
      else:
        stage_from(ys[0])

      ysrc = ysh
    else:
      ysrc = ys[0]
    pltpu.sync_copy(zer_hbm, rows0_v)
    for z in range(ZCH):
      pltpu.sync_copy(rows0_v, acc_sh.at[pl.ds(sid * (ZCH * K) + z * K, K)])
    plsc.subcore_barrier()

    def sidx(c):
      return src_v.at[pl.ds(c * K, K)]

    pltpu.async_copy(ysrc.at[sidx(0)], rows0_v, sem0)

    def body(c2, carry):
      c = 2 * c2
      pltpu.make_async_copy(ysrc.at[sidx(c)], rows0_v, sem0).wait()
      pltpu.async_copy(ysrc.at[sidx(c + 1)], rows1_v, sem1)
      pltpu.sync_copy(rows0_v, acc_sh.at[dst_v.at[c]], add=True)
      pltpu.make_async_copy(ysrc.at[sidx(c + 1)], rows1_v, sem1).wait()

      @pl.when(c + 2 < nch)
      def _():
        pltpu.async_copy(ysrc.at[sidx(c + 2)], rows0_v, sem0)

      pltpu.sync_copy(rows1_v, acc_sh.at[dst_v.at[c + 1]], add=True)
      return carry

    lax.fori_loop(0, nch // 2, body, 0)
    plsc.subcore_barrier()
    off = 0
    for w in OCH:
      r0 = sid * STRIPE + off
      pltpu.sync_copy(acc_sh.at[pl.ds(r0, w)], rows0_v.at[pl.ds(0, w)])
      pltpu.sync_copy(rows0_v.at[pl.ds(0, w)],
                      out_hbm.at[pl.ds(r0, w), pl.ds(cid * F, F)])
      off += w

    @pl.when(sid == NS - 1)
    def _():
      t0 = NS * STRIPE
      pltpu.sync_copy(acc_sh.at[pl.ds(t0, TAIL)], rows0_v.at[pl.ds(0, TAIL)])
      pltpu.sync_copy(rows0_v.at[pl.ds(0, TAIL)],
                      out_hbm.at[pl.ds(t0, TAIL), pl.ds(cid * F, F)])

  kern = functools.partial(
      pl.kernel,
      out_type=jax.ShapeDtypeStruct((N, 128), jnp.float32),
      mesh=mesh,
      scratch_types=scratch,
      compiler_params=pltpu.CompilerParams(use_tc_tiling_on_sc=False),
  )

  if col_split:
    @kern
    def sc(ya, yb, *rest):
      body_fn((ya, yb), *rest[:11], rest[11:])
  else:
    @kern
    def sc(y, *rest):
      body_fn((y,), *rest[:11], rest[11:])

  return sc


def _mm4_body(x_ref, wr_ref, wt_ref, ya_ref, yb_ref, ra_ref, rb_ref):
  xb = x_ref[...]
  dn = (((1,), (1,)), ((), ()))
  ya_ref[...] = lax.dot_general(xb, wr_ref[0], dn,
                                preferred_element_type=jnp.float32)
  yb_ref[...] = lax.dot_general(xb, wr_ref[1], dn,
                                preferred_element_type=jnp.float32)
  ra_ref[...] = lax.dot_general(xb, wt_ref[0], dn,
                                preferred_element_type=jnp.float32)
  rb_ref[...] = lax.dot_general(xb, wt_ref[1], dn,
                                preferred_element_type=jnp.float32)


def _mm4(x, w_rel, w_root):
  fin = x.shape[1]
  fh = w_rel.shape[0] // 2
  grid = N // BR
  row = pl.BlockSpec((BR, fh), lambda i: (i, 0))
  wsp = pl.BlockSpec((2, fh, fin), lambda i: (0, 0, 0))
  osh = jax.ShapeDtypeStruct((N, fh), jnp.float32)
  return pl.pallas_call(
      _mm4_body,
      grid=(grid,),
      in_specs=[pl.BlockSpec((BR, fin), lambda i: (i, 0)), wsp, wsp],
      out_specs=[row, row, row, row],
      out_shape=[osh, osh, osh, osh],
  )(x, w_rel.reshape(2, fh, fin), w_root.reshape(2, fh, fin))


def _bn_tail(z, i, zbuf, s_acc):
  zbuf[pl.ds(i * BR, BR), :] = z
  cs = jnp.sum(z, axis=0, keepdims=True)
  cq = jnp.sum(z * z, axis=0, keepdims=True)
  blk = jnp.concatenate([cs, cq, jnp.zeros((6, z.shape[1]), jnp.float32)],
                        axis=0)

  @pl.when(i == 0)
  def _():
    s_acc[...] = blk

  @pl.when(i != 0)
  def _():
    s_acc[...] = s_acc[...] + blk


def _bn_mm_phase1(i, g_ref, be_ref, wr_ref, wt_ref, y_ref, r_ref,
                  zbuf, s_acc):
  s = s_acc[...]
  mu = s[0:1, :] * (1.0 / N)
  var = s[1:2, :] * (1.0 / N) - mu * mu
  scale = g_ref[...] * lax.rsqrt(var + 1e-5)
  shift = be_ref[...] - mu * scale
  h = jnp.maximum(zbuf[pl.ds(i * BR, BR), :] * scale + shift, 0.0)
  dn = (((1,), (1,)), ((), ()))
  y_ref[...] = lax.dot_general(h, wr_ref[...], dn,
                               preferred_element_type=jnp.float32)
  r_ref[...] = lax.dot_general(h, wt_ref[...], dn,
                               preferred_element_type=jnp.float32)


def _fused_cat_body(p_ref, ra_ref, rb_ref, g_ref, be_ref,
                    wr_ref, wt_ref, y_ref, r_ref, zbuf, s_acc):
  t = pl.program_id(0)
  i = pl.program_id(1)

  @pl.when(t == 0)
  def _():
    z = p_ref[...] + jnp.concatenate([ra_ref[...], rb_ref[...]], axis=1)
    _bn_tail(z, i, zbuf, s_acc)

  @pl.when(t == 1)
  def _():
    _bn_mm_phase1(i, g_ref, be_ref, wr_ref, wt_ref, y_ref, r_ref,
                  zbuf, s_acc)


def _fused_cat(p, ra, rb, g, be, w_rel, w_root):
  fin = 2 * ra.shape[1]
  fh = ra.shape[1]
  fout = w_rel.shape[0]
  nb = N // BR
  prow = pl.BlockSpec((BR, fin), lambda t, i: (i * (1 - t), 0))
  row = pl.BlockSpec((BR, fh), lambda t, i: (i * (1 - t), 0))
  vec = lambda f: pl.BlockSpec((1, f), lambda t, i: (0, 0))
  wsp = pl.BlockSpec((fout, fin), lambda t, i: (0, 0))
  orow = pl.BlockSpec((BR, fout), lambda t, i: (i * t, 0))
  return pl.pallas_call(
      _fused_cat_body,
      grid=(2, nb),
      in_specs=[prow, row, row, vec(fin), vec(fin), wsp, wsp],
      out_specs=[orow, orow],
      out_shape=[
          jax.ShapeDtypeStruct((N, fout), jnp.float32),
          jax.ShapeDtypeStruct((N, fout), jnp.float32),
      ],
      scratch_shapes=[
          pltpu.VMEM((N, fin), jnp.float32),
          pltpu.VMEM((8, fin), jnp.float32),
      ],
  )(p, ra, rb, g.reshape(1, fin), be.reshape(1, fin), w_rel, w_root)


def _fused_sum_body(p_ref, r_ref, g_ref, be_ref,
                    wr_ref, wt_ref, y_ref, r_out_ref, zbuf, s_acc):
  t = pl.program_id(0)
  i = pl.program_id(1)

  @pl.when(t == 0)
  def _():
    pb = p_ref[...]
    fin = r_ref.shape[1]
    z = pb[:, :fin] + pb[:, fin:2 * fin] + r_ref[...]
    _bn_tail(z, i, zbuf, s_acc)

  @pl.when(t == 1)
  def _():
    _bn_mm_phase1(i, g_ref, be_ref, wr_ref, wt_ref, y_ref, r_out_ref,
                  zbuf, s_acc)


def _fused_sum(p, r, g, be, w_rel, w_root):
  fin = r.shape[1]
  fout = w_rel.shape[0]
  nb = N // BR
  prow = pl.BlockSpec((BR, 128), lambda t, i: (i * (1 - t), 0))
  row = pl.BlockSpec((BR, fin), lambda t, i: (i * (1 - t), 0))
  vec = pl.BlockSpec((1, fin), lambda t, i: (0, 0))
  wsp = pl.BlockSpec((fout, fin), lambda t, i: (0, 0))
  orow = pl.BlockSpec((BR, fout), lambda t, i: (i * t, 0))
  return pl.pallas_call(
      _fused_sum_body,
      grid=(2, nb),
      in_specs=[prow, row, vec, vec, wsp, wsp],
      out_specs=[orow, orow],
      out_shape=[
          jax.ShapeDtypeStruct((N, fout), jnp.float32),
          jax.ShapeDtypeStruct((N, fout), jnp.float32),
      ],
      scratch_shapes=[
          pltpu.VMEM((N, fin), jnp.float32),
          pltpu.VMEM((8, fin), jnp.float32),
      ],
  )(p, r, g.reshape(1, fin), be.reshape(1, fin), w_rel, w_root)


def _final_body(p_ref, r_ref, b3_ref, bt_ref, wl_ref, bl_ref,
                out_ref, acc_ref):
  i = pl.program_id(0)

  @pl.when(i == 0)
  def _():
    acc_ref[...] = jnp.zeros_like(acc_ref)

  pb = p_ref[...]
  h = jnp.maximum(pb[:, :32] + pb[:, 32:64] + r_ref[...] + b3_ref[...], 0.0)
  bt = bt_ref[0, 0, :]
  oh = (lax.broadcasted_iota(jnp.int32, (G, BR), 0) == bt[None, :]).astype(
      jnp.float32)
  hcat = jnp.concatenate(
      [h, jnp.ones((BR, 1), jnp.float32), jnp.zeros((BR, 31), jnp.float32)],
      axis=1)
  acc_ref[...] += lax.dot_general(oh, hcat, (((1,), (0,)), ((), ())),
                                  preferred_element_type=jnp.float32)

  @pl.when(i == pl.num_programs(0) - 1)
  def _():
    acc = acc_ref[...]
    pooled = acc[:, :32] / jnp.maximum(acc[:, 32:33], 1.0)
    logits = lax.dot_general(pooled, wl_ref[...], (((1,), (1,)), ((), ())),
                             preferred_element_type=jnp.float32)
    logits = logits + bl_ref[...]
    m = jnp.max(logits, axis=1, keepdims=True)
    e = jnp.exp(logits - m)
    out_ref[...] = logits - m - jnp.log(jnp.sum(e, axis=1, keepdims=True))


def _final(p, r3, b3, batch3, wl, bl):
  grid = N // BR
  return pl.pallas_call(
      _final_body,
      grid=(grid,),
      in_specs=[
          pl.BlockSpec((BR, 128), lambda i: (i, 0)),
          pl.BlockSpec((BR, 32), lambda i: (i, 0)),
          pl.BlockSpec((1, 32), lambda i: (0, 0)),
          pl.BlockSpec((1, 1, BR), lambda i: (i, 0, 0)),
          pl.BlockSpec((C, 32), lambda i: (0, 0)),
          pl.BlockSpec((1, C), lambda i: (0, 0)),
      ],
      out_specs=pl.BlockSpec((G, C), lambda i: (0, 0)),
      out_shape=jax.ShapeDtypeStruct((G, C), jnp.float32),
      scratch_shapes=[pltpu.VMEM((G, 64), jnp.float32)],
  )(p, r3, b3.reshape(1, 32), batch3, wl, bl.reshape(1, C))


def kernel(x, edge_index, batch, W1_rel, b1, W1_root, g1, be1, W2_rel, b2,
           W2_root, g2, be2, W3_rel, b3, W3_root, Wl, bl):
  src = edge_index[0]
  dst = edge_index[1]
  pad = E_PAD - E
  srcf = jnp.concatenate([src, jnp.zeros((pad,), jnp.int32)])
  dstm = jnp.concatenate([dst, jnp.full((pad,), N, jnp.int32)]).reshape(
      NC * NS, CH, K)

  dstm16 = dstm.reshape(NS, 2 * CH, K)

  sc64cs = _sc_segment_sum(64, col_split=True)
  sc64 = _sc_segment_sum(64, stage_y=True)
  sc32 = _sc_segment_sum(32, stage_y=True)
  z64 = jnp.zeros((K, 64), jnp.float32)
  z32 = jnp.zeros((K, 32), jnp.float32)

  ya, yb, ra, rb = _mm4(x, W1_rel, W1_root)
  p1 = sc64cs(ya, yb, srcf, dstm16, z64)
  y2, r2 = _fused_cat(p1, ra, rb, g1, be1, W2_rel, W2_root)
  p2 = sc64(y2, srcf, dstm, z64)
  y3, r3 = _fused_sum(p2, r2, g2, be2, W3_rel, W3_root)
  p3 = sc32(y3, srcf, dstm, z32)
  batch3 = batch.reshape(N // BR, 1, BR)
  return _final(p3, r3, b3, batch3, Wl, bl)


# speedup vs baseline: 1.3128x; 1.0937x over previous
import functools

import jax
import jax.numpy as jnp
from jax import lax
from jax.experimental import pallas as pl
from jax.experimental.pallas import tpu as pltpu
from jax.experimental.pallas import tpu_sc as plsc

N = 10000
E = 160000
G = 64
C = 36

NC = 2
NS = 16
K = 128
CH = 40
E_PAD = NC * NS * CH * K
NPAD = 10240
ZCH = NPAD // NS // K
STRIPE = 624
TAIL = N - NS * STRIPE
OCH = [K] * 4 + [STRIPE - 4 * K]

BR = 2000


def _sc_segment_sum(F, stage_y=False, col_split=False):
  nch = 2 * CH if col_split else CH
  stage_y = stage_y or col_split
  mesh = plsc.VectorSubcoreMesh(core_axis_name="c", subcore_axis_name="s",
                                num_cores=NC, num_subcores=NS)
  scratch = [
      pltpu.VMEM((nch * K,), jnp.int32),
      pltpu.VMEM((nch, K), jnp.int32),
      pltpu.VMEM((K, F), jnp.float32),
      pltpu.VMEM((K, F), jnp.float32),
      pltpu.VMEM_SHARED((NPAD, F), jnp.float32),
      pltpu.SemaphoreType.DMA,
      pltpu.SemaphoreType.DMA,
  ]
  if stage_y:
    scratch.append(pltpu.VMEM_SHARED((N, F), jnp.float32))

  def body_fn(y_hbm, srcf_hbm, dstm_hbm, zer_hbm, out_hbm,
              src_v, dst_v, rows0_v, rows1_v, acc_sh, sem0, sem1, *maybe_ysh):
    cid = lax.axis_index("c")
    sid = lax.axis_index("s")
    wid = sid if col_split else cid * NS + sid
    pltpu.sync_copy(srcf_hbm.at[pl.ds(wid * nch * K, nch * K)], src_v)
    pltpu.sync_copy(dstm_hbm.at[wid], dst_v)
    if stage_y:
      ysh = maybe_ysh[0]
      col0 = cid * F if col_split else 0
      off = 0
      for w in OCH:
        r0 = sid * STRIPE + off
        pltpu.sync_copy(y_hbm.at[pl.ds(r0, w), pl.ds(col0, F)],
                        rows1_v.at[pl.ds(0, w)])
        pltpu.sync_copy(rows1_v.at[pl.ds(0, w)], ysh.at[pl.ds(r0, w)])
        off += w

      @pl.when(sid == NS - 1)
      def _():
        t0 = NS * STRIPE
        pltpu.sync_copy(y_hbm.at[pl.ds(t0, TAIL), pl.ds(col0, F)],
                        rows1_v.at[pl.ds(0, TAIL)])
        pltpu.sync_copy(rows1_v.at[pl.ds(0, TAIL)], ysh.at[pl.ds(t0, TAIL)])

      ysrc = ysh
    else:
      ysrc = y_hbm
    pltpu.sync_copy(zer_hbm, rows0_v)
    for z in range(ZCH):
      pltpu.sync_copy(rows0_v, acc_sh.at[pl.ds(sid * (ZCH * K) + z * K, K)])
    plsc.subcore_barrier()

    def sidx(c):
      return src_v.at[pl.ds(c * K, K)]

    pltpu.async_copy(ysrc.at[sidx(0)], rows0_v, sem0)

    def body(c2, carry):
      c = 2 * c2
      pltpu.make_async_copy(ysrc.at[sidx(c)], rows0_v, sem0).wait()
      pltpu.async_copy(ysrc.at[sidx(c + 1)], rows1_v, sem1)
      pltpu.sync_copy(rows0_v, acc_sh.at[dst_v.at[c]], add=True)
      pltpu.make_async_copy(ysrc.at[sidx(c + 1)], rows1_v, sem1).wait()

      @pl.when(c + 2 < nch)
      def _():
        pltpu.async_copy(ysrc.at[sidx(c + 2)], rows0_v, sem0)

      pltpu.sync_copy(rows1_v, acc_sh.at[dst_v.at[c + 1]], add=True)
      return carry

    lax.fori_loop(0, nch // 2, body, 0)
    plsc.subcore_barrier()
    off = 0
    for w in OCH:
      r0 = sid * STRIPE + off
      pltpu.sync_copy(acc_sh.at[pl.ds(r0, w)], rows0_v.at[pl.ds(0, w)])
      pltpu.sync_copy(rows0_v.at[pl.ds(0, w)],
                      out_hbm.at[pl.ds(r0, w), pl.ds(cid * F, F)])
      off += w

    @pl.when(sid == NS - 1)
    def _():
      t0 = NS * STRIPE
      pltpu.sync_copy(acc_sh.at[pl.ds(t0, TAIL)], rows0_v.at[pl.ds(0, TAIL)])
      pltpu.sync_copy(rows0_v.at[pl.ds(0, TAIL)],
                      out_hbm.at[pl.ds(t0, TAIL), pl.ds(cid * F, F)])

  kern = functools.partial(
      pl.kernel,
      out_type=jax.ShapeDtypeStruct((N, 128), jnp.float32),
      mesh=mesh,
      scratch_types=scratch,
      compiler_params=pltpu.CompilerParams(use_tc_tiling_on_sc=False),
  )

  sc = kern(body_fn)
  return sc


def _mm2_body(x_ref, wr_ref, wt_ref, y_ref, r_ref):
  xb = x_ref[...]
  dn = (((1,), (1,)), ((), ()))

  def two(w3):
    return jnp.concatenate(
        [lax.dot_general(xb, w3[0], dn, preferred_element_type=jnp.float32),
         lax.dot_general(xb, w3[1], dn, preferred_element_type=jnp.float32)],
        axis=1)

  y_ref[...] = two(wr_ref[...])
  r_ref[...] = two(wt_ref[...])


def _mm2(x, w_rel, w_root):
  fin = x.shape[1]
  fh = w_rel.shape[0] // 2
  grid = N // BR
  row = pl.BlockSpec((BR, 2 * fh), lambda i: (i, 0))
  wsp = pl.BlockSpec((2, fh, fin), lambda i: (0, 0, 0))
  osh = jax.ShapeDtypeStruct((N, 2 * fh), jnp.float32)
  return pl.pallas_call(
      _mm2_body,
      grid=(grid,),
      in_specs=[pl.BlockSpec((BR, fin), lambda i: (i, 0)), wsp, wsp],
      out_specs=[row, row],
      out_shape=[osh, osh],
  )(x, w_rel.reshape(2, fh, fin), w_root.reshape(2, fh, fin))


def _fused_body(cat, fin, p_ref, r_ref, g_ref, be_ref, wr_ref, wt_ref,
                out_ref, zbuf, s_acc):
  t = pl.program_id(0)
  i = pl.program_id(1)

  @pl.when(t == 0)
  def _():
    if cat:
      z = p_ref[...] + r_ref[...]
    else:
      pb = p_ref[...]
      z = pb[:, :fin] + pb[:, fin:2 * fin] + r_ref[...][:, fin:2 * fin]
    zbuf[pl.ds(i * BR, BR), :] = z
    cs = jnp.sum(z, axis=0, keepdims=True)
    cq = jnp.sum(z * z, axis=0, keepdims=True)
    blk = jnp.concatenate([cs, cq, jnp.zeros((6, fin), jnp.float32)], axis=0)

    @pl.when(i == 0)
    def _():
      s_acc[...] = blk

    @pl.when(i != 0)
    def _():
      s_acc[...] = s_acc[...] + blk

  @pl.when(t == 1)
  def _():
    s = s_acc[...]
    mu = s[0:1, :] * (1.0 / N)
    var = s[1:2, :] * (1.0 / N) - mu * mu
    scale = g_ref[...] * lax.rsqrt(var + 1e-5)
    shift = be_ref[...] - mu * scale
    h = jnp.maximum(zbuf[pl.ds(i * BR, BR), :] * scale + shift, 0.0)
    dn = (((1,), (1,)), ((), ()))
    fout = wr_ref.shape[0]
    parts = [
        lax.dot_general(h, wr_ref[...], dn, preferred_element_type=jnp.float32),
        lax.dot_general(h, wt_ref[...], dn, preferred_element_type=jnp.float32),
    ]
    if 2 * fout < 128:
      parts.append(jnp.zeros((BR, 128 - 2 * fout), jnp.float32))
    out_ref[...] = jnp.concatenate(parts, axis=1)


def _fused(p, rsrc, g, be, w_rel, w_root, cat):
  fin = w_rel.shape[1]
  fout = w_rel.shape[0]
  nb = N // BR
  prow = pl.BlockSpec((BR, 128), lambda t, i: (i * (1 - t), 0))
  rrow = pl.BlockSpec((BR, 128), lambda t, i: (i * (1 - t), 0))
  vec = pl.BlockSpec((1, fin), lambda t, i: (0, 0))
  wsp = pl.BlockSpec((fout, fin), lambda t, i: (0, 0))
  orow = pl.BlockSpec((BR, 128), lambda t, i: (i * t, 0))
  return pl.pallas_call(
      functools.partial(_fused_body, cat, fin),
      grid=(2, nb),
      in_specs=[prow, rrow, vec, vec, wsp, wsp],
      out_specs=orow,
      out_shape=jax.ShapeDtypeStruct((N, 128), jnp.float32),
      scratch_shapes=[
          pltpu.VMEM((N, fin), jnp.float32),
          pltpu.VMEM((8, fin), jnp.float32),
      ],
  )(p, rsrc, g.reshape(1, fin), be.reshape(1, fin), w_rel, w_root)


def _final_body(p_ref, r_ref, b3_ref, bt_ref, wl_ref, bl_ref,
                out_ref, acc_ref):
  i = pl.program_id(0)

  @pl.when(i == 0)
  def _():
    acc_ref[...] = jnp.zeros_like(acc_ref)

  pb = p_ref[...]
  h = jnp.maximum(
      pb[:, :32] + pb[:, 32:64] + r_ref[...][:, 32:64] + b3_ref[...], 0.0)
  bt = bt_ref[0, 0, :]
  oh = (lax.broadcasted_iota(jnp.int32, (G, BR), 0) == bt[None, :]).astype(
      jnp.float32)
  hcat = jnp.concatenate(
      [h, jnp.ones((BR, 1), jnp.float32), jnp.zeros((BR, 31), jnp.float32)],
      axis=1)
  acc_ref[...] += lax.dot_general(oh, hcat, (((1,), (0,)), ((), ())),
                                  preferred_element_type=jnp.float32)

  @pl.when(i == pl.num_programs(0) - 1)
  def _():
    acc = acc_ref[...]
    pooled = acc[:, :32] / jnp.maximum(acc[:, 32:33], 1.0)
    logits = lax.dot_general(pooled, wl_ref[...], (((1,), (1,)), ((), ())),
                             preferred_element_type=jnp.float32)
    logits = logits + bl_ref[...]
    m = jnp.max(logits, axis=1, keepdims=True)
    e = jnp.exp(logits - m)
    out_ref[...] = logits - m - jnp.log(jnp.sum(e, axis=1, keepdims=True))


def _final(p, r3, b3, batch3, wl, bl):
  grid = N // BR
  return pl.pallas_call(
      _final_body,
      grid=(grid,),
      in_specs=[
          pl.BlockSpec((BR, 128), lambda i: (i, 0)),
          pl.BlockSpec((BR, 128), lambda i: (i, 0)),
          pl.BlockSpec((1, 32), lambda i: (0, 0)),
          pl.BlockSpec((1, 1, BR), lambda i: (i, 0, 0)),
          pl.BlockSpec((C, 32), lambda i: (0, 0)),
          pl.BlockSpec((1, C), lambda i: (0, 0)),
      ],
      out_specs=pl.BlockSpec((G, C), lambda i: (0, 0)),
      out_shape=jax.ShapeDtypeStruct((G, C), jnp.float32),
      scratch_shapes=[pltpu.VMEM((G, 64), jnp.float32)],
  )(p, r3, b3.reshape(1, 32), batch3, wl, bl.reshape(1, C))


def kernel(x, edge_index, batch, W1_rel, b1, W1_root, g1, be1, W2_rel, b2,
           W2_root, g2, be2, W3_rel, b3, W3_root, Wl, bl):
  src = edge_index[0]
  dst = edge_index[1]
  pad = E_PAD - E
  srcf = jnp.concatenate([src, jnp.zeros((pad,), jnp.int32)])
  dstm = jnp.concatenate([dst, jnp.full((pad,), N, jnp.int32)]).reshape(
      NC * NS, CH, K)

  dstm16 = dstm.reshape(NS, 2 * CH, K)

  sc64cs = _sc_segment_sum(64, col_split=True)
  sc64 = _sc_segment_sum(64, stage_y=True)
  sc32 = _sc_segment_sum(32, stage_y=True)
  z64 = jnp.zeros((K, 64), jnp.float32)
  z32 = jnp.zeros((K, 32), jnp.float32)

  ycat, rcat = _mm2(x, W1_rel, W1_root)
  p1 = sc64cs(ycat, srcf, dstm16, z64)
  y2r2 = _fused(p1, rcat, g1, be1, W2_rel, W2_root, cat=True)
  p2 = sc64(y2r2, srcf, dstm, z64)
  y3r3 = _fused(p2, y2r2, g2, be2, W3_rel, W3_root, cat=False)
  p3 = sc32(y3r3, srcf, dstm, z32)
  batch3 = batch.reshape(N // BR, 1, BR)
  return _final(p3, y3r3, b3, batch3, Wl, bl)


# speedup vs baseline: 1.3349x; 1.0168x over previous
import functools

import jax
import jax.numpy as jnp
from jax import lax
from jax.experimental import pallas as pl
from jax.experimental.pallas import tpu as pltpu
from jax.experimental.pallas import tpu_sc as plsc

N = 10000
E = 160000
G = 64
C = 36

NC = 2
NS = 16
K = 128
CH = 40
E_PAD = NC * NS * CH * K
NPAD = 10240
ZCH = NPAD // NS // K
STRIPE = 624
TAIL = N - NS * STRIPE
OCH = [K] * 4 + [STRIPE - 4 * K]

BR = 2000


def _sc_segment_sum(F, stage_y=False, col_split=False):
  nch = 2 * CH if col_split else CH
  nbuf = 2 if col_split else 4
  stage_y = stage_y or col_split
  mesh = plsc.VectorSubcoreMesh(core_axis_name="c", subcore_axis_name="s",
                                num_cores=NC, num_subcores=NS)
  scratch = (
      [pltpu.VMEM((nch * K,), jnp.int32), pltpu.VMEM((nch, K), jnp.int32)]
      + [pltpu.VMEM((K, F), jnp.float32)] * nbuf
      + [pltpu.VMEM_SHARED((NPAD, F), jnp.float32)]
      + [pltpu.SemaphoreType.DMA] * nbuf
  )
  if stage_y:
    scratch.append(pltpu.VMEM_SHARED((N, F), jnp.float32))

  def body_fn(y_hbm, srcf_hbm, dstm_hbm, zer_hbm, out_hbm,
              src_v, dst_v, *rest):
    bufs = rest[:nbuf]
    acc_sh = rest[nbuf]
    sems = rest[nbuf + 1:2 * nbuf + 1]
    maybe_ysh = rest[2 * nbuf + 1:]
    rows0_v, rows1_v = bufs[0], bufs[1]
    cid = lax.axis_index("c")
    sid = lax.axis_index("s")
    wid = sid if col_split else cid * NS + sid
    pltpu.sync_copy(srcf_hbm.at[pl.ds(wid * nch * K, nch * K)], src_v)
    pltpu.sync_copy(dstm_hbm.at[wid], dst_v)
    if stage_y:
      ysh = maybe_ysh[0]
      col0 = cid * F if col_split else 0
      off = 0
      for w in OCH:
        r0 = sid * STRIPE + off
        pltpu.sync_copy(y_hbm.at[pl.ds(r0, w), pl.ds(col0, F)],
                        rows1_v.at[pl.ds(0, w)])
        pltpu.sync_copy(rows1_v.at[pl.ds(0, w)], ysh.at[pl.ds(r0, w)])
        off += w

      @pl.when(sid == NS - 1)
      def _():
        t0 = NS * STRIPE
        pltpu.sync_copy(y_hbm.at[pl.ds(t0, TAIL), pl.ds(col0, F)],
                        rows1_v.at[pl.ds(0, TAIL)])
        pltpu.sync_copy(rows1_v.at[pl.ds(0, TAIL)], ysh.at[pl.ds(t0, TAIL)])

      ysrc = ysh
    else:
      ysrc = y_hbm
    pltpu.sync_copy(zer_hbm, rows0_v)
    for z in range(ZCH):
      pltpu.sync_copy(rows0_v, acc_sh.at[pl.ds(sid * (ZCH * K) + z * K, K)])
    plsc.subcore_barrier()

    def sidx(c):
      return src_v.at[pl.ds(c * K, K)]

    for j in range(nbuf):
      pltpu.async_copy(ysrc.at[sidx(j)], bufs[j], sems[j])

    def body(cg, carry):
      c = nbuf * cg
      for j in range(nbuf):
        pltpu.make_async_copy(ysrc.at[sidx(c + j)], bufs[j], sems[j]).wait()
        pltpu.sync_copy(bufs[j], acc_sh.at[dst_v.at[c + j]], add=True)

        @pl.when(c + j + nbuf < nch)
        def _():
          pltpu.async_copy(ysrc.at[sidx(c + j + nbuf)], bufs[j], sems[j])

      return carry

    lax.fori_loop(0, nch // nbuf, body, 0)
    plsc.subcore_barrier()
    off = 0
    for w in OCH:
      r0 = sid * STRIPE + off
      pltpu.sync_copy(acc_sh.at[pl.ds(r0, w)], rows0_v.at[pl.ds(0, w)])
      pltpu.sync_copy(rows0_v.at[pl.ds(0, w)],
                      out_hbm.at[pl.ds(r0, w), pl.ds(cid * F, F)])
      off += w

    @pl.when(sid == NS - 1)
    def _():
      t0 = NS * STRIPE
      pltpu.sync_copy(acc_sh.at[pl.ds(t0, TAIL)], rows0_v.at[pl.ds(0, TAIL)])
      pltpu.sync_copy(rows0_v.at[pl.ds(0, TAIL)],
                      out_hbm.at[pl.ds(t0, TAIL), pl.ds(cid * F, F)])

  kern = functools.partial(
      pl.kernel,
      out_type=jax.ShapeDtypeStruct((N, 128), jnp.float32),
      mesh=mesh,
      scratch_types=scratch,
      compiler_params=pltpu.CompilerParams(use_tc_tiling_on_sc=False),
  )

  sc = kern(body_fn)
  return sc


def _mm2_body(x_ref, wr_ref, wt_ref, y_ref, r_ref):
  xb = x_ref[...]
  dn = (((1,), (1,)), ((), ()))

  def two(w3):
    return jnp.concatenate(
        [lax.dot_general(xb, w3[0], dn, preferred_element_type=jnp.float32),
         lax.dot_general(xb, w3[1], dn, preferred_element_type=jnp.float32)],
        axis=1)

  y_ref[...] = two(wr_ref[...])
  r_ref[...] = two(wt_ref[...])


def _mm2(x, w_rel, w_root):
  fin = x.shape[1]
  fh = w_rel.shape[0] // 2
  grid = N // BR
  row = pl.BlockSpec((BR, 2 * fh), lambda i: (i, 0))
  wsp = pl.BlockSpec((2, fh, fin), lambda i: (0, 0, 0))
  osh = jax.ShapeDtypeStruct((N, 2 * fh), jnp.float32)
  return pl.pallas_call(
      _mm2_body,
      grid=(grid,),
      in_specs=[pl.BlockSpec((BR, fin), lambda i: (i, 0)), wsp, wsp],
      out_specs=[row, row],
      out_shape=[osh, osh],
  )(x, w_rel.reshape(2, fh, fin), w_root.reshape(2, fh, fin))


def _fused_body(cat, fin, p_ref, r_ref, g_ref, be_ref, wr_ref, wt_ref,
                out_ref, zbuf, s_acc):
  t = pl.program_id(0)
  i = pl.program_id(1)

  @pl.when(t == 0)
  def _():
    if cat:
      z = p_ref[...] + r_ref[...]
    else:
      pb = p_ref[...]
      z = pb[:, :fin] + pb[:, fin:2 * fin] + r_ref[...][:, fin:2 * fin]
    zbuf[pl.ds(i * BR, BR), :] = z
    cs = jnp.sum(z, axis=0, keepdims=True)
    cq = jnp.sum(z * z, axis=0, keepdims=True)
    blk = jnp.concatenate([cs, cq, jnp.zeros((6, fin), jnp.float32)], axis=0)

    @pl.when(i == 0)
    def _():
      s_acc[...] = blk

    @pl.when(i != 0)
    def _():
      s_acc[...] = s_acc[...] + blk

  @pl.when(t == 1)
  def _():
    s = s_acc[...]
    mu = s[0:1, :] * (1.0 / N)
    var = s[1:2, :] * (1.0 / N) - mu * mu
    scale = g_ref[...] * lax.rsqrt(var + 1e-5)
    shift = be_ref[...] - mu * scale
    h = jnp.maximum(zbuf[pl.ds(i * BR, BR), :] * scale + shift, 0.0)
    dn = (((1,), (1,)), ((), ()))
    fout = wr_ref.shape[0]
    parts = [
        lax.dot_general(h, wr_ref[...], dn, preferred_element_type=jnp.float32),
        lax.dot_general(h, wt_ref[...], dn, preferred_element_type=jnp.float32),
    ]
    if 2 * fout < 128:
      parts.append(jnp.zeros((BR, 128 - 2 * fout), jnp.float32))
    out_ref[...] = jnp.concatenate(parts, axis=1)


def _fused(p, rsrc, g, be, w_rel, w_root, cat):
  fin = w_rel.shape[1]
  fout = w_rel.shape[0]
  nb = N // BR
  prow = pl.BlockSpec((BR, 128), lambda t, i: (i * (1 - t), 0))
  rrow = pl.BlockSpec((BR, 128), lambda t, i: (i * (1 - t), 0))
  vec = pl.BlockSpec((1, fin), lambda t, i: (0, 0))
  wsp = pl.BlockSpec((fout, fin), lambda t, i: (0, 0))
  orow = pl.BlockSpec((BR, 128), lambda t, i: (i * t, 0))
  return pl.pallas_call(
      functools.partial(_fused_body, cat, fin),
      grid=(2, nb),
      in_specs=[prow, rrow, vec, vec, wsp, wsp],
      out_specs=orow,
      out_shape=jax.ShapeDtypeStruct((N, 128), jnp.float32),
      scratch_shapes=[
          pltpu.VMEM((N, fin), jnp.float32),
          pltpu.VMEM((8, fin), jnp.float32),
      ],
  )(p, rsrc, g.reshape(1, fin), be.reshape(1, fin), w_rel, w_root)


def _final_body(p_ref, r_ref, b3_ref, bt_ref, wl_ref, bl_ref,
                out_ref, acc_ref):
  i = pl.program_id(0)

  @pl.when(i == 0)
  def _():
    acc_ref[...] = jnp.zeros_like(acc_ref)

  pb = p_ref[...]
  h = jnp.maximum(
      pb[:, :32] + pb[:, 32:64] + r_ref[...][:, 32:64] + b3_ref[...], 0.0)
  bt = bt_ref[0, 0, :]
  oh = (lax.broadcasted_iota(jnp.int32, (G, BR), 0) == bt[None, :]).astype(
      jnp.float32)
  hcat = jnp.concatenate(
      [h, jnp.ones((BR, 1), jnp.float32), jnp.zeros((BR, 31), jnp.float32)],
      axis=1)
  acc_ref[...] += lax.dot_general(oh, hcat, (((1,), (0,)), ((), ())),
                                  preferred_element_type=jnp.float32)

  @pl.when(i == pl.num_programs(0) - 1)
  def _():
    acc = acc_ref[...]
    pooled = acc[:, :32] / jnp.maximum(acc[:, 32:33], 1.0)
    logits = lax.dot_general(pooled, wl_ref[...], (((1,), (1,)), ((), ())),
                             preferred_element_type=jnp.float32)
    logits = logits + bl_ref[...]
    m = jnp.max(logits, axis=1, keepdims=True)
    e = jnp.exp(logits - m)
    out_ref[...] = logits - m - jnp.log(jnp.sum(e, axis=1, keepdims=True))


def _final(p, r3, b3, batch3, wl, bl):
  grid = N // BR
  return pl.pallas_call(
      _final_body,
      grid=(grid,),
      in_specs=[
          pl.BlockSpec((BR, 128), lambda i: (i, 0)),
          pl.BlockSpec((BR, 128), lambda i: (i, 0)),
          pl.BlockSpec((1, 32), lambda i: (0, 0)),
          pl.BlockSpec((1, 1, BR), lambda i: (i, 0, 0)),
          pl.BlockSpec((C, 32), lambda i: (0, 0)),
          pl.BlockSpec((1, C), lambda i: (0, 0)),
      ],
      out_specs=pl.BlockSpec((G, C), lambda i: (0, 0)),
      out_shape=jax.ShapeDtypeStruct((G, C), jnp.float32),
      scratch_shapes=[pltpu.VMEM((G, 64), jnp.float32)],
  )(p, r3, b3.reshape(1, 32), batch3, wl, bl.reshape(1, C))


def kernel(x, edge_index, batch, W1_rel, b1, W1_root, g1, be1, W2_rel, b2,
           W2_root, g2, be2, W3_rel, b3, W3_root, Wl, bl):
  src = edge_index[0]
  dst = edge_index[1]
  pad = E_PAD - E
  srcf = jnp.concatenate([src, jnp.zeros((pad,), jnp.int32)])
  dstm = jnp.concatenate([dst, jnp.full((pad,), N, jnp.int32)]).reshape(
      NC * NS, CH, K)

  dstm16 = dstm.reshape(NS, 2 * CH, K)

  sc64cs = _sc_segment_sum(64, col_split=True)
  sc64 = _sc_segment_sum(64, stage_y=True)
  sc32 = _sc_segment_sum(32, stage_y=True)
  z64 = jnp.zeros((K, 64), jnp.float32)
  z32 = jnp.zeros((K, 32), jnp.float32)

  ycat, rcat = _mm2(x, W1_rel, W1_root)
  p1 = sc64cs(ycat, srcf, dstm16, z64)
  y2r2 = _fused(p1, rcat, g1, be1, W2_rel, W2_root, cat=True)
  p2 = sc64(y2r2, srcf, dstm, z64)
  y3r3 = _fused(p2, y2r2, g2, be2, W3_rel, W3_root, cat=False)
  p3 = sc32(y3r3, srcf, dstm, z32)
  batch3 = batch.reshape(N // BR, 1, BR)
  return _final(p3, y3r3, b3, batch3, Wl, bl)
